# Initial kernel scaffold; baseline (speedup 1.0000x reference)
import jax
import jax.numpy as jnp
from jax.experimental import pallas as pl


def kernel(feature, memory, selected_neg_idx):
    raise NotImplementedError("write your pallas kernel here")


# speedup vs baseline: 20.0958x; 20.0958x over previous
import functools

import jax
import jax.numpy as jnp
from jax import lax
from jax.experimental import pallas as pl
from jax.experimental.pallas import tpu as pltpu
from jax.experimental.pallas import tpu_sc as plsc

B = 128
D = 128
M = 100000
K = 4096

NC = 2
NS = 16
NW = NC * NS
ROWS_PER_TILE = B // NW

MC = 4096


def _sims_body(feat_ref, mem_ref, out_ref):
    out_ref[...] = lax.dot_general(
        feat_ref[...], mem_ref[...],
        dimension_numbers=(((1,), (1,)), ((), ())),
        preferred_element_type=jnp.float32,
    )


def _sims(feature, memory):
    n_blocks = (M + MC - 1) // MC
    return pl.pallas_call(
        _sims_body,
        grid=(n_blocks,),
        in_specs=[
            pl.BlockSpec((B, D), lambda i: (0, 0)),
            pl.BlockSpec((MC, D), lambda i: (i, 0)),
        ],
        out_specs=pl.BlockSpec((B, MC), lambda i: (0, i)),
        out_shape=jax.ShapeDtypeStruct((B, M), jnp.float32),
    )(feature, memory)


_mesh = plsc.VectorSubcoreMesh(core_axis_name="c", subcore_axis_name="s")


@functools.partial(
    pl.kernel,
    out_type=jax.ShapeDtypeStruct((B, K), jnp.float32),
    mesh=_mesh,
    compiler_params=pltpu.CompilerParams(needs_layout_passes=False),
    scratch_types=[
        pltpu.VMEM((M,), jnp.float32),
        pltpu.VMEM((K,), jnp.int32),
        pltpu.VMEM((K,), jnp.float32),
    ],
)
def _gather(sims_hbm, idx_hbm, out_hbm, sims_v, idx_v, out_v):
    wid = lax.axis_index("s") * NC + lax.axis_index("c")
    for j in range(ROWS_PER_TILE):
        b = wid * ROWS_PER_TILE + j
        pltpu.sync_copy(idx_hbm.at[b], idx_v)
        pltpu.sync_copy(sims_hbm.at[b], sims_v)

        def body(t, _):
            iv = idx_v[pl.ds(t * 16, 16)]
            out_v[pl.ds(t * 16, 16)] = plsc.load_gather(sims_v, [iv])
            return 0

        lax.fori_loop(0, K // 16, body, 0)
        pltpu.sync_copy(out_v, out_hbm.at[b])


def kernel(feature, memory, selected_neg_idx):
    idx = selected_neg_idx.reshape(B, K).astype(jnp.int32)
    sims = _sims(feature, memory)
    out = _gather(sims, idx)
    return out[..., None]
